# Initial kernel scaffold; baseline (speedup 1.0000x reference)
import jax
import jax.numpy as jnp
from jax.experimental import pallas as pl


def kernel(x, edge_index, batch, W1, b1, W2, b2):
    raise NotImplementedError("write your pallas kernel here")


# speedup vs baseline: 24.6255x; 24.6255x over previous
import functools
import jax
import jax.numpy as jnp
from jax import lax
from jax.experimental import pallas as pl
from jax.experimental.pallas import tpu as pltpu
from jax.experimental.pallas import tpu_sc as plsc

N = 10000
E = 320000
IN_D = 128
HID = 64
OUT_D = 116
NC, NS = 2, 16
NW = NC * NS
K = 100
CPT = 100
N_PAD = 10240
RPT = N_PAD // NS
DEG_D = 16

_mesh = plsc.VectorSubcoreMesh(
    core_axis_name="c", subcore_axis_name="s", num_cores=NC, num_subcores=NS
)


@functools.partial(
    pl.kernel,
    out_type=jax.ShapeDtypeStruct((NC, N_PAD, DEG_D), jnp.float32),
    mesh=_mesh,
    compiler_params=pltpu.CompilerParams(use_tc_tiling_on_sc=False),
    scratch_types=[
        pltpu.VMEM((CPT, K), jnp.int32),
        pltpu.VMEM((K, DEG_D), jnp.float32),
        pltpu.VMEM_SHARED((N_PAD, DEG_D), jnp.float32),
    ],
)
def _deg_kernel(edges, ones_hbm, zeros_hbm, out, dslab, ones_v, acc):
    ci = lax.axis_index("c")
    si = lax.axis_index("s")
    wid = ci * NS + si
    r0 = si * RPT
    pltpu.sync_copy(ones_hbm, ones_v)
    pltpu.sync_copy(edges.at[1, wid], dslab)
    pltpu.sync_copy(zeros_hbm.at[pl.ds(r0, RPT)], acc.at[pl.ds(r0, RPT)])
    plsc.subcore_barrier()

    def body(c, carry):
        pltpu.sync_copy(ones_v, acc.at[dslab.at[c]], add=True)
        return carry

    lax.fori_loop(0, CPT, body, 0)
    plsc.subcore_barrier()
    pltpu.sync_copy(acc.at[pl.ds(r0, RPT)], out.at[ci, pl.ds(r0, RPT)])


@functools.partial(
    pl.kernel,
    out_type=jax.ShapeDtypeStruct((NC, N_PAD, HID), jnp.float32),
    mesh=_mesh,
    compiler_params=pltpu.CompilerParams(use_tc_tiling_on_sc=False),
    scratch_types=[
        pltpu.VMEM((CPT, K), jnp.int32),
        pltpu.VMEM((CPT, K), jnp.int32),
        pltpu.VMEM((K, HID), jnp.float32),
        pltpu.VMEM_SHARED((N_PAD, HID), jnp.float32),
        pltpu.SemaphoreType.DMA,
    ],
)
def _prop_kernel(table, edges, zeros_hbm, out, sslab, dslab, rows, acc, sem):
    ci = lax.axis_index("c")
    si = lax.axis_index("s")
    wid = ci * NS + si
    r0 = si * RPT
    pltpu.sync_copy(edges.at[0, wid], sslab)
    pltpu.sync_copy(edges.at[1, wid], dslab)
    pltpu.sync_copy(zeros_hbm.at[pl.ds(r0, RPT)], acc.at[pl.ds(r0, RPT)])
    plsc.subcore_barrier()

    def body(c, carry):
        pltpu.async_copy(table.at[sslab.at[c]], rows, sem).wait()
        pltpu.sync_copy(rows, acc.at[dslab.at[c]], add=True)
        return carry

    lax.fori_loop(0, CPT, body, 0)
    plsc.subcore_barrier()
    pltpu.sync_copy(acc.at[pl.ds(r0, RPT)], out.at[ci, pl.ds(r0, RPT)])


BM = 1000


def _mm1_body(x_ref, w_ref, da_ref, db_ref, g_ref, dis_ref):
    deg = 1.0 + da_ref[:, 0:1] + db_ref[:, 0:1]
    dis = lax.rsqrt(deg)
    h = jnp.dot(x_ref[:], w_ref[:], preferred_element_type=jnp.float32)
    g_ref[:] = h * dis
    dis_ref[:] = dis


def _mid_body(g_ref, aa_ref, ab_ref, dis_ref, b1_ref, o_ref):
    dis = dis_ref[:]
    z = (g_ref[:] + aa_ref[:] + ab_ref[:]) * dis + b1_ref[:]
    o_ref[:] = jnp.maximum(z, 0.0) * dis


def _mm2_body(g_ref, aa_ref, ab_ref, dis_ref, w_ref, b2_ref, o_ref):
    u = (g_ref[:] + aa_ref[:] + ab_ref[:]) * dis_ref[:]
    o_ref[:] = (
        jnp.dot(u, w_ref[:], preferred_element_type=jnp.float32) + b2_ref[:]
    )


def kernel(x, edge_index, batch, W1, b1, W2, b2):
    del batch
    edges = edge_index.reshape(2, NW, CPT, K)
    ones16 = jnp.ones((K, DEG_D), jnp.float32)
    zeros_deg = jnp.zeros((N_PAD, DEG_D), jnp.float32)
    zeros_hid = jnp.zeros((N_PAD, HID), jnp.float32)

    deg = _deg_kernel(edges, ones16, zeros_deg)

    g1, dis = pl.pallas_call(
        _mm1_body,
        grid=(N // BM,),
        in_specs=[
            pl.BlockSpec((BM, IN_D), lambda i: (i, 0)),
            pl.BlockSpec((IN_D, HID), lambda i: (0, 0)),
            pl.BlockSpec((BM, DEG_D), lambda i: (i, 0)),
            pl.BlockSpec((BM, DEG_D), lambda i: (i, 0)),
        ],
        out_specs=[
            pl.BlockSpec((BM, HID), lambda i: (i, 0)),
            pl.BlockSpec((BM, 1), lambda i: (i, 0)),
        ],
        out_shape=[
            jax.ShapeDtypeStruct((N, HID), jnp.float32),
            jax.ShapeDtypeStruct((N, 1), jnp.float32),
        ],
    )(x, W1, deg[0], deg[1])

    acc1 = _prop_kernel(g1, edges, zeros_hid)

    g2 = pl.pallas_call(
        _mid_body,
        grid=(N // BM,),
        in_specs=[
            pl.BlockSpec((BM, HID), lambda i: (i, 0)),
            pl.BlockSpec((BM, HID), lambda i: (i, 0)),
            pl.BlockSpec((BM, HID), lambda i: (i, 0)),
            pl.BlockSpec((BM, 1), lambda i: (i, 0)),
            pl.BlockSpec((1, HID), lambda i: (0, 0)),
        ],
        out_specs=pl.BlockSpec((BM, HID), lambda i: (i, 0)),
        out_shape=jax.ShapeDtypeStruct((N, HID), jnp.float32),
    )(g1, acc1[0], acc1[1], dis, b1.reshape(1, HID))

    acc2 = _prop_kernel(g2, edges, zeros_hid)

    out = pl.pallas_call(
        _mm2_body,
        grid=(N // BM,),
        in_specs=[
            pl.BlockSpec((BM, HID), lambda i: (i, 0)),
            pl.BlockSpec((BM, HID), lambda i: (i, 0)),
            pl.BlockSpec((BM, HID), lambda i: (i, 0)),
            pl.BlockSpec((BM, 1), lambda i: (i, 0)),
            pl.BlockSpec((HID, OUT_D), lambda i: (0, 0)),
            pl.BlockSpec((1, OUT_D), lambda i: (0, 0)),
        ],
        out_specs=pl.BlockSpec((BM, OUT_D), lambda i: (i, 0)),
        out_shape=jax.ShapeDtypeStruct((N, OUT_D), jnp.float32),
    )(g2, acc2[0], acc2[1], dis, W2, b2.reshape(1, OUT_D))

    return out


# speedup vs baseline: 28.7481x; 1.1674x over previous
import functools
import jax
import jax.numpy as jnp
from jax import lax
from jax.experimental import pallas as pl
from jax.experimental.pallas import tpu as pltpu
from jax.experimental.pallas import tpu_sc as plsc

N = 10000
E = 320000
IN_D = 128
HID = 64
OUT_D = 116
NC, NS = 2, 16
NW = NC * NS
K = 100
CPT = 100
N_PAD = 10240
RPT = N_PAD // NS
DEG_D = 16

_mesh = plsc.VectorSubcoreMesh(
    core_axis_name="c", subcore_axis_name="s", num_cores=NC, num_subcores=NS
)


@functools.partial(
    pl.kernel,
    out_type=jax.ShapeDtypeStruct((NC, N_PAD, DEG_D), jnp.float32),
    mesh=_mesh,
    compiler_params=pltpu.CompilerParams(use_tc_tiling_on_sc=False),
    scratch_types=[
        pltpu.VMEM((CPT, K), jnp.int32),
        pltpu.VMEM((K, DEG_D), jnp.float32),
        pltpu.VMEM_SHARED((N_PAD, DEG_D), jnp.float32),
    ],
)
def _deg_kernel(edges, ones_hbm, zeros_hbm, out, dslab, ones_v, acc):
    ci = lax.axis_index("c")
    si = lax.axis_index("s")
    wid = ci * NS + si
    r0 = si * RPT
    pltpu.sync_copy(ones_hbm, ones_v)
    pltpu.sync_copy(edges.at[1, wid], dslab)
    pltpu.sync_copy(zeros_hbm.at[pl.ds(r0, RPT)], acc.at[pl.ds(r0, RPT)])
    plsc.subcore_barrier()

    def body(c, carry):
        pltpu.sync_copy(ones_v, acc.at[dslab.at[c]], add=True)
        return carry

    lax.fori_loop(0, CPT, body, 0)
    plsc.subcore_barrier()
    pltpu.sync_copy(acc.at[pl.ds(r0, RPT)], out.at[ci, pl.ds(r0, RPT)])


@functools.partial(
    pl.kernel,
    out_type=jax.ShapeDtypeStruct((NC, N_PAD, HID), jnp.float32),
    mesh=_mesh,
    compiler_params=pltpu.CompilerParams(use_tc_tiling_on_sc=False),
    scratch_types=[
        pltpu.VMEM((CPT, K), jnp.int32),
        pltpu.VMEM((CPT, K), jnp.int32),
        pltpu.VMEM((K, HID), jnp.float32),
        pltpu.VMEM((K, HID), jnp.float32),
        pltpu.VMEM_SHARED((N_PAD, HID), jnp.float32),
        pltpu.SemaphoreType.DMA,
        pltpu.SemaphoreType.DMA,
    ],
)
def _prop_kernel(
    table, edges, zeros_hbm, out, sslab, dslab, rows0, rows1, acc, sem0, sem1
):
    ci = lax.axis_index("c")
    si = lax.axis_index("s")
    wid = ci * NS + si
    r0 = si * RPT
    pltpu.sync_copy(edges.at[0, wid], sslab)
    pltpu.sync_copy(edges.at[1, wid], dslab)
    pltpu.async_copy(table.at[sslab.at[0]], rows0, sem0)
    pltpu.sync_copy(zeros_hbm.at[pl.ds(r0, RPT)], acc.at[pl.ds(r0, RPT)])
    plsc.subcore_barrier()

    def body(j, carry):
        c0 = 2 * j
        pltpu.make_async_copy(table.at[sslab.at[c0]], rows0, sem0).wait()
        pltpu.async_copy(table.at[sslab.at[c0 + 1]], rows1, sem1)
        pltpu.sync_copy(rows0, acc.at[dslab.at[c0]], add=True)
        pltpu.make_async_copy(table.at[sslab.at[c0 + 1]], rows1, sem1).wait()

        @pl.when(j + 1 < CPT // 2)
        def _():
            pltpu.async_copy(table.at[sslab.at[c0 + 2]], rows0, sem0)

        pltpu.sync_copy(rows1, acc.at[dslab.at[c0 + 1]], add=True)
        return carry

    lax.fori_loop(0, CPT // 2, body, 0)
    plsc.subcore_barrier()
    pltpu.sync_copy(acc.at[pl.ds(r0, RPT)], out.at[ci, pl.ds(r0, RPT)])


BM = 1000


def _mm1_body(x_ref, w_ref, da_ref, db_ref, g_ref, dis_ref):
    deg = 1.0 + da_ref[:, 0:1] + db_ref[:, 0:1]
    dis = lax.rsqrt(deg)
    h = jnp.dot(x_ref[:], w_ref[:], preferred_element_type=jnp.float32)
    g_ref[:] = h * dis
    dis_ref[:] = dis


def _mid_body(g_ref, aa_ref, ab_ref, dis_ref, b1_ref, o_ref):
    dis = dis_ref[:]
    z = (g_ref[:] + aa_ref[:] + ab_ref[:]) * dis + b1_ref[:]
    o_ref[:] = jnp.maximum(z, 0.0) * dis


def _mm2_body(g_ref, aa_ref, ab_ref, dis_ref, w_ref, b2_ref, o_ref):
    u = (g_ref[:] + aa_ref[:] + ab_ref[:]) * dis_ref[:]
    o_ref[:] = (
        jnp.dot(u, w_ref[:], preferred_element_type=jnp.float32) + b2_ref[:]
    )


def kernel(x, edge_index, batch, W1, b1, W2, b2):
    del batch
    edges = edge_index.reshape(2, NW, CPT, K)
    ones16 = jnp.ones((K, DEG_D), jnp.float32)
    zeros_deg = jnp.zeros((N_PAD, DEG_D), jnp.float32)
    zeros_hid = jnp.zeros((N_PAD, HID), jnp.float32)

    deg = _deg_kernel(edges, ones16, zeros_deg)

    g1, dis = pl.pallas_call(
        _mm1_body,
        grid=(N // BM,),
        in_specs=[
            pl.BlockSpec((BM, IN_D), lambda i: (i, 0)),
            pl.BlockSpec((IN_D, HID), lambda i: (0, 0)),
            pl.BlockSpec((BM, DEG_D), lambda i: (i, 0)),
            pl.BlockSpec((BM, DEG_D), lambda i: (i, 0)),
        ],
        out_specs=[
            pl.BlockSpec((BM, HID), lambda i: (i, 0)),
            pl.BlockSpec((BM, 1), lambda i: (i, 0)),
        ],
        out_shape=[
            jax.ShapeDtypeStruct((N, HID), jnp.float32),
            jax.ShapeDtypeStruct((N, 1), jnp.float32),
        ],
    )(x, W1, deg[0], deg[1])

    acc1 = _prop_kernel(g1, edges, zeros_hid)

    g2 = pl.pallas_call(
        _mid_body,
        grid=(N // BM,),
        in_specs=[
            pl.BlockSpec((BM, HID), lambda i: (i, 0)),
            pl.BlockSpec((BM, HID), lambda i: (i, 0)),
            pl.BlockSpec((BM, HID), lambda i: (i, 0)),
            pl.BlockSpec((BM, 1), lambda i: (i, 0)),
            pl.BlockSpec((1, HID), lambda i: (0, 0)),
        ],
        out_specs=pl.BlockSpec((BM, HID), lambda i: (i, 0)),
        out_shape=jax.ShapeDtypeStruct((N, HID), jnp.float32),
    )(g1, acc1[0], acc1[1], dis, b1.reshape(1, HID))

    acc2 = _prop_kernel(g2, edges, zeros_hid)

    out = pl.pallas_call(
        _mm2_body,
        grid=(N // BM,),
        in_specs=[
            pl.BlockSpec((BM, HID), lambda i: (i, 0)),
            pl.BlockSpec((BM, HID), lambda i: (i, 0)),
            pl.BlockSpec((BM, HID), lambda i: (i, 0)),
            pl.BlockSpec((BM, 1), lambda i: (i, 0)),
            pl.BlockSpec((HID, OUT_D), lambda i: (0, 0)),
            pl.BlockSpec((1, OUT_D), lambda i: (0, 0)),
        ],
        out_specs=pl.BlockSpec((BM, OUT_D), lambda i: (i, 0)),
        out_shape=jax.ShapeDtypeStruct((N, OUT_D), jnp.float32),
    )(g2, acc2[0], acc2[1], dis, W2, b2.reshape(1, OUT_D))

    return out


# speedup vs baseline: 37.2901x; 1.2971x over previous
import functools
import jax
import jax.numpy as jnp
from jax import lax
from jax.experimental import pallas as pl
from jax.experimental.pallas import tpu as pltpu
from jax.experimental.pallas import tpu_sc as plsc

N = 10000
E = 320000
IN_D = 128
HID = 64
OUT_D = 116
NC, NS = 2, 16
NW = NC * NS
K = 100
CPT = 100
N_PAD = 10240
RPT = N_PAD // NS
DEG_D = 16

_mesh = plsc.VectorSubcoreMesh(
    core_axis_name="c", subcore_axis_name="s", num_cores=NC, num_subcores=NS
)


@functools.partial(
    pl.kernel,
    out_type=jax.ShapeDtypeStruct((NC, N_PAD, DEG_D), jnp.float32),
    mesh=_mesh,
    compiler_params=pltpu.CompilerParams(use_tc_tiling_on_sc=False),
    scratch_types=[
        pltpu.VMEM((CPT, K), jnp.int32),
        pltpu.VMEM((K, DEG_D), jnp.float32),
        pltpu.VMEM_SHARED((N_PAD, DEG_D), jnp.float32),
    ],
)
def _deg_kernel(edges, ones_hbm, zeros_hbm, out, dslab, ones_v, acc):
    ci = lax.axis_index("c")
    si = lax.axis_index("s")
    wid = ci * NS + si
    r0 = si * RPT
    pltpu.sync_copy(ones_hbm, ones_v)
    pltpu.sync_copy(edges.at[1, wid], dslab)
    pltpu.sync_copy(zeros_hbm.at[pl.ds(r0, RPT)], acc.at[pl.ds(r0, RPT)])
    plsc.subcore_barrier()

    def body(c, carry):
        pltpu.sync_copy(ones_v, acc.at[dslab.at[c]], add=True)
        return carry

    lax.fori_loop(0, CPT, body, 0)
    plsc.subcore_barrier()
    pltpu.sync_copy(acc.at[pl.ds(r0, RPT)], out.at[ci, pl.ds(r0, RPT)])


@functools.partial(
    pl.kernel,
    out_type=jax.ShapeDtypeStruct((NC, N_PAD, HID), jnp.float32),
    mesh=_mesh,
    compiler_params=pltpu.CompilerParams(use_tc_tiling_on_sc=False),
    scratch_types=[
        pltpu.VMEM((CPT, K), jnp.int32),
        pltpu.VMEM((CPT, K), jnp.int32),
        [pltpu.VMEM((K, HID), jnp.float32)] * 4,
        pltpu.VMEM_SHARED((N_PAD, HID), jnp.float32),
        [pltpu.SemaphoreType.DMA] * 4,
        [pltpu.SemaphoreType.DMA] * 4,
    ],
)
def _prop_kernel(
    table, edges, zeros_hbm, out, sslab, dslab, rows, acc, gsem, ssem
):
    ci = lax.axis_index("c")
    si = lax.axis_index("s")
    wid = ci * NS + si
    r0 = si * RPT
    pltpu.sync_copy(edges.at[0, wid], sslab)
    pltpu.sync_copy(edges.at[1, wid], dslab)
    for b in range(4):
        pltpu.async_copy(table.at[sslab.at[b]], rows[b], gsem[b])
    pltpu.sync_copy(zeros_hbm.at[pl.ds(r0, RPT)], acc.at[pl.ds(r0, RPT)])
    plsc.subcore_barrier()

    def body(j, carry):
        c0 = 4 * j
        for b in range(4):
            pltpu.make_async_copy(
                table.at[sslab.at[c0 + b]], rows[b], gsem[b]
            ).wait()
            pltpu.async_copy(
                rows[b], acc.at[dslab.at[c0 + b]], ssem[b], add=True
            )
        for b in range(4):

            @pl.when(c0 + b + 4 < CPT)
            def _(b=b):
                pltpu.make_async_copy(
                    rows[b], acc.at[dslab.at[c0 + b]], ssem[b]
                ).wait()
                pltpu.async_copy(
                    table.at[sslab.at[c0 + b + 4]], rows[b], gsem[b]
                )

        return carry

    lax.fori_loop(0, CPT // 4, body, 0)
    for b in range(4):
        pltpu.make_async_copy(
            rows[b], acc.at[dslab.at[CPT - 4 + b]], ssem[b]
        ).wait()
    plsc.subcore_barrier()
    pltpu.sync_copy(acc.at[pl.ds(r0, RPT)], out.at[ci, pl.ds(r0, RPT)])


BM = 1000


def _mm1_body(x_ref, w_ref, da_ref, db_ref, g_ref, dis_ref):
    deg = 1.0 + da_ref[:, 0:1] + db_ref[:, 0:1]
    dis = lax.rsqrt(deg)
    h = jnp.dot(x_ref[:], w_ref[:], preferred_element_type=jnp.float32)
    g_ref[:] = h * dis
    dis_ref[:] = dis


def _mid_body(g_ref, aa_ref, ab_ref, dis_ref, b1_ref, o_ref):
    dis = dis_ref[:]
    z = (g_ref[:] + aa_ref[:] + ab_ref[:]) * dis + b1_ref[:]
    o_ref[:] = jnp.maximum(z, 0.0) * dis


def _mm2_body(g_ref, aa_ref, ab_ref, dis_ref, w_ref, b2_ref, o_ref):
    u = (g_ref[:] + aa_ref[:] + ab_ref[:]) * dis_ref[:]
    o_ref[:] = (
        jnp.dot(u, w_ref[:], preferred_element_type=jnp.float32) + b2_ref[:]
    )


def kernel(x, edge_index, batch, W1, b1, W2, b2):
    del batch
    edges = edge_index.reshape(2, NW, CPT, K)
    ones16 = jnp.ones((K, DEG_D), jnp.float32)
    zeros_deg = jnp.zeros((N_PAD, DEG_D), jnp.float32)
    zeros_hid = jnp.zeros((N_PAD, HID), jnp.float32)

    deg = _deg_kernel(edges, ones16, zeros_deg)

    g1, dis = pl.pallas_call(
        _mm1_body,
        grid=(N // BM,),
        in_specs=[
            pl.BlockSpec((BM, IN_D), lambda i: (i, 0)),
            pl.BlockSpec((IN_D, HID), lambda i: (0, 0)),
            pl.BlockSpec((BM, DEG_D), lambda i: (i, 0)),
            pl.BlockSpec((BM, DEG_D), lambda i: (i, 0)),
        ],
        out_specs=[
            pl.BlockSpec((BM, HID), lambda i: (i, 0)),
            pl.BlockSpec((BM, 1), lambda i: (i, 0)),
        ],
        out_shape=[
            jax.ShapeDtypeStruct((N, HID), jnp.float32),
            jax.ShapeDtypeStruct((N, 1), jnp.float32),
        ],
    )(x, W1, deg[0], deg[1])

    acc1 = _prop_kernel(g1, edges, zeros_hid)

    g2 = pl.pallas_call(
        _mid_body,
        grid=(N // BM,),
        in_specs=[
            pl.BlockSpec((BM, HID), lambda i: (i, 0)),
            pl.BlockSpec((BM, HID), lambda i: (i, 0)),
            pl.BlockSpec((BM, HID), lambda i: (i, 0)),
            pl.BlockSpec((BM, 1), lambda i: (i, 0)),
            pl.BlockSpec((1, HID), lambda i: (0, 0)),
        ],
        out_specs=pl.BlockSpec((BM, HID), lambda i: (i, 0)),
        out_shape=jax.ShapeDtypeStruct((N, HID), jnp.float32),
    )(g1, acc1[0], acc1[1], dis, b1.reshape(1, HID))

    acc2 = _prop_kernel(g2, edges, zeros_hid)

    out = pl.pallas_call(
        _mm2_body,
        grid=(N // BM,),
        in_specs=[
            pl.BlockSpec((BM, HID), lambda i: (i, 0)),
            pl.BlockSpec((BM, HID), lambda i: (i, 0)),
            pl.BlockSpec((BM, HID), lambda i: (i, 0)),
            pl.BlockSpec((BM, 1), lambda i: (i, 0)),
            pl.BlockSpec((HID, OUT_D), lambda i: (0, 0)),
            pl.BlockSpec((1, OUT_D), lambda i: (0, 0)),
        ],
        out_specs=pl.BlockSpec((BM, OUT_D), lambda i: (i, 0)),
        out_shape=jax.ShapeDtypeStruct((N, OUT_D), jnp.float32),
    )(g2, acc2[0], acc2[1], dis, W2, b2.reshape(1, OUT_D))

    return out


# speedup vs baseline: 38.7455x; 1.0390x over previous
import functools
import jax
import jax.numpy as jnp
from jax import lax
from jax.experimental import pallas as pl
from jax.experimental.pallas import tpu as pltpu
from jax.experimental.pallas import tpu_sc as plsc

N = 10000
E = 320000
IN_D = 128
HID = 64
OUT_D = 116
NC, NS = 2, 16
NW = NC * NS
K = 250
CPT = 40
N_PAD = 10240
RPT = N_PAD // NS
DEG_D = 16

_mesh = plsc.VectorSubcoreMesh(
    core_axis_name="c", subcore_axis_name="s", num_cores=NC, num_subcores=NS
)


@functools.partial(
    pl.kernel,
    out_type=jax.ShapeDtypeStruct((NC, N_PAD, DEG_D), jnp.float32),
    mesh=_mesh,
    compiler_params=pltpu.CompilerParams(use_tc_tiling_on_sc=False),
    scratch_types=[
        pltpu.VMEM((CPT, K), jnp.int32),
        pltpu.VMEM((K, DEG_D), jnp.float32),
        pltpu.VMEM_SHARED((N_PAD, DEG_D), jnp.float32),
    ],
)
def _deg_kernel(edges, ones_hbm, zeros_hbm, out, dslab, ones_v, acc):
    ci = lax.axis_index("c")
    si = lax.axis_index("s")
    wid = ci * NS + si
    r0 = si * RPT
    pltpu.sync_copy(ones_hbm, ones_v)
    pltpu.sync_copy(edges.at[1, wid], dslab)
    pltpu.sync_copy(zeros_hbm.at[pl.ds(r0, RPT)], acc.at[pl.ds(r0, RPT)])
    plsc.subcore_barrier()

    def body(c, carry):
        pltpu.sync_copy(ones_v, acc.at[dslab.at[c]], add=True)
        return carry

    lax.fori_loop(0, CPT, body, 0)
    plsc.subcore_barrier()
    pltpu.sync_copy(acc.at[pl.ds(r0, RPT)], out.at[ci, pl.ds(r0, RPT)])


@functools.partial(
    pl.kernel,
    out_type=jax.ShapeDtypeStruct((NC, N_PAD, HID), jnp.float32),
    mesh=_mesh,
    compiler_params=pltpu.CompilerParams(use_tc_tiling_on_sc=False),
    scratch_types=[
        pltpu.VMEM((CPT, K), jnp.int32),
        pltpu.VMEM((CPT, K), jnp.int32),
        [pltpu.VMEM((K, HID), jnp.float32)] * 4,
        pltpu.VMEM_SHARED((N_PAD, HID), jnp.float32),
        [pltpu.SemaphoreType.DMA] * 4,
        [pltpu.SemaphoreType.DMA] * 4,
    ],
)
def _prop_kernel(
    table, edges, zeros_hbm, out, sslab, dslab, rows, acc, gsem, ssem
):
    ci = lax.axis_index("c")
    si = lax.axis_index("s")
    wid = ci * NS + si
    r0 = si * RPT
    pltpu.sync_copy(edges.at[0, wid], sslab)
    pltpu.sync_copy(edges.at[1, wid], dslab)
    for b in range(4):
        pltpu.async_copy(table.at[sslab.at[b]], rows[b], gsem[b])
    pltpu.sync_copy(zeros_hbm.at[pl.ds(r0, RPT)], acc.at[pl.ds(r0, RPT)])
    plsc.subcore_barrier()

    def body(j, carry):
        c0 = 4 * j
        for b in range(4):
            pltpu.make_async_copy(
                table.at[sslab.at[c0 + b]], rows[b], gsem[b]
            ).wait()
            pltpu.async_copy(
                rows[b], acc.at[dslab.at[c0 + b]], ssem[b], add=True
            )
        for b in range(4):

            @pl.when(c0 + b + 4 < CPT)
            def _(b=b):
                pltpu.make_async_copy(
                    rows[b], acc.at[dslab.at[c0 + b]], ssem[b]
                ).wait()
                pltpu.async_copy(
                    table.at[sslab.at[c0 + b + 4]], rows[b], gsem[b]
                )

        return carry

    lax.fori_loop(0, CPT // 4, body, 0)
    for b in range(4):
        pltpu.make_async_copy(
            rows[b], acc.at[dslab.at[CPT - 4 + b]], ssem[b]
        ).wait()
    plsc.subcore_barrier()
    pltpu.sync_copy(acc.at[pl.ds(r0, RPT)], out.at[ci, pl.ds(r0, RPT)])


BM = 1000


def _mm1_body(x_ref, w_ref, da_ref, db_ref, g_ref, dis_ref):
    deg = 1.0 + da_ref[:, 0:1] + db_ref[:, 0:1]
    dis = lax.rsqrt(deg)
    h = jnp.dot(x_ref[:], w_ref[:], preferred_element_type=jnp.float32)
    g_ref[:] = h * dis
    dis_ref[:] = dis


def _mid_body(g_ref, aa_ref, ab_ref, dis_ref, b1_ref, o_ref):
    dis = dis_ref[:]
    z = (g_ref[:] + aa_ref[:] + ab_ref[:]) * dis + b1_ref[:]
    o_ref[:] = jnp.maximum(z, 0.0) * dis


def _mm2_body(g_ref, aa_ref, ab_ref, dis_ref, w_ref, b2_ref, o_ref):
    u = (g_ref[:] + aa_ref[:] + ab_ref[:]) * dis_ref[:]
    o_ref[:] = (
        jnp.dot(u, w_ref[:], preferred_element_type=jnp.float32) + b2_ref[:]
    )


def kernel(x, edge_index, batch, W1, b1, W2, b2):
    del batch
    edges = edge_index.reshape(2, NW, CPT, K)
    ones16 = jnp.ones((K, DEG_D), jnp.float32)
    zeros_deg = jnp.zeros((N_PAD, DEG_D), jnp.float32)
    zeros_hid = jnp.zeros((N_PAD, HID), jnp.float32)

    deg = _deg_kernel(edges, ones16, zeros_deg)

    g1, dis = pl.pallas_call(
        _mm1_body,
        grid=(N // BM,),
        in_specs=[
            pl.BlockSpec((BM, IN_D), lambda i: (i, 0)),
            pl.BlockSpec((IN_D, HID), lambda i: (0, 0)),
            pl.BlockSpec((BM, DEG_D), lambda i: (i, 0)),
            pl.BlockSpec((BM, DEG_D), lambda i: (i, 0)),
        ],
        out_specs=[
            pl.BlockSpec((BM, HID), lambda i: (i, 0)),
            pl.BlockSpec((BM, 1), lambda i: (i, 0)),
        ],
        out_shape=[
            jax.ShapeDtypeStruct((N, HID), jnp.float32),
            jax.ShapeDtypeStruct((N, 1), jnp.float32),
        ],
    )(x, W1, deg[0], deg[1])

    acc1 = _prop_kernel(g1, edges, zeros_hid)

    g2 = pl.pallas_call(
        _mid_body,
        grid=(N // BM,),
        in_specs=[
            pl.BlockSpec((BM, HID), lambda i: (i, 0)),
            pl.BlockSpec((BM, HID), lambda i: (i, 0)),
            pl.BlockSpec((BM, HID), lambda i: (i, 0)),
            pl.BlockSpec((BM, 1), lambda i: (i, 0)),
            pl.BlockSpec((1, HID), lambda i: (0, 0)),
        ],
        out_specs=pl.BlockSpec((BM, HID), lambda i: (i, 0)),
        out_shape=jax.ShapeDtypeStruct((N, HID), jnp.float32),
    )(g1, acc1[0], acc1[1], dis, b1.reshape(1, HID))

    acc2 = _prop_kernel(g2, edges, zeros_hid)

    out = pl.pallas_call(
        _mm2_body,
        grid=(N // BM,),
        in_specs=[
            pl.BlockSpec((BM, HID), lambda i: (i, 0)),
            pl.BlockSpec((BM, HID), lambda i: (i, 0)),
            pl.BlockSpec((BM, HID), lambda i: (i, 0)),
            pl.BlockSpec((BM, 1), lambda i: (i, 0)),
            pl.BlockSpec((HID, OUT_D), lambda i: (0, 0)),
            pl.BlockSpec((1, OUT_D), lambda i: (0, 0)),
        ],
        out_specs=pl.BlockSpec((BM, OUT_D), lambda i: (i, 0)),
        out_shape=jax.ShapeDtypeStruct((N, OUT_D), jnp.float32),
    )(g2, acc2[0], acc2[1], dis, W2, b2.reshape(1, OUT_D))

    return out


# speedup vs baseline: 43.2258x; 1.1156x over previous
import functools
import jax
import jax.numpy as jnp
from jax import lax
from jax.experimental import pallas as pl
from jax.experimental.pallas import tpu as pltpu
from jax.experimental.pallas import tpu_sc as plsc

N = 10000
E = 320000
IN_D = 128
HID = 64
OUT_D = 116
NC, NS = 2, 16
NW = NC * NS
K = 200
CPT = 50
EPT = CPT * K
N_PAD = 10240
RPT = N_PAD // NS
DEG_D = 16

_mesh = plsc.VectorSubcoreMesh(
    core_axis_name="c", subcore_axis_name="s", num_cores=NC, num_subcores=NS
)


@functools.partial(
    pl.kernel,
    out_type=jax.ShapeDtypeStruct((NC, N_PAD, DEG_D), jnp.float32),
    mesh=_mesh,
    compiler_params=pltpu.CompilerParams(use_tc_tiling_on_sc=False),
    scratch_types=[
        pltpu.VMEM((EPT,), jnp.int32),
        pltpu.VMEM((K, DEG_D), jnp.float32),
        pltpu.VMEM_SHARED((N_PAD, DEG_D), jnp.float32),
    ],
)
def _deg_kernel(edges, ones_hbm, zeros_hbm, out, dslab, ones_v, acc):
    ci = lax.axis_index("c")
    si = lax.axis_index("s")
    wid = ci * NS + si
    r0 = si * RPT
    pltpu.sync_copy(ones_hbm, ones_v)
    pltpu.sync_copy(edges.at[1, pl.ds(wid * EPT, EPT)], dslab)
    pltpu.sync_copy(zeros_hbm.at[pl.ds(r0, RPT)], acc.at[pl.ds(r0, RPT)])
    plsc.subcore_barrier()

    def body(c, carry):
        pltpu.sync_copy(ones_v, acc.at[dslab.at[pl.ds(c * K, K)]], add=True)
        return carry

    lax.fori_loop(0, CPT, body, 0)
    plsc.subcore_barrier()
    pltpu.sync_copy(acc.at[pl.ds(r0, RPT)], out.at[ci, pl.ds(r0, RPT)])


@functools.partial(
    pl.kernel,
    out_type=jax.ShapeDtypeStruct((NC, N_PAD, HID), jnp.float32),
    mesh=_mesh,
    compiler_params=pltpu.CompilerParams(use_tc_tiling_on_sc=False),
    scratch_types=[
        pltpu.VMEM((EPT,), jnp.int32),
        pltpu.VMEM((EPT,), jnp.int32),
        [pltpu.VMEM((K, HID), jnp.float32)] * 4,
        pltpu.VMEM_SHARED((N_PAD, HID), jnp.float32),
        [pltpu.SemaphoreType.DMA] * 4,
        [pltpu.SemaphoreType.DMA] * 4,
    ],
)
def _prop_kernel(
    table, edges, zeros_hbm, out, sslab, dslab, rows, acc, gsem, ssem
):
    ci = lax.axis_index("c")
    si = lax.axis_index("s")
    wid = ci * NS + si
    r0 = si * RPT
    pltpu.sync_copy(edges.at[0, pl.ds(wid * EPT, EPT)], sslab)
    pltpu.sync_copy(edges.at[1, pl.ds(wid * EPT, EPT)], dslab)
    for b in range(4):
        pltpu.async_copy(
            table.at[sslab.at[pl.ds(b * K, K)]], rows[b], gsem[b]
        )
    pltpu.sync_copy(zeros_hbm.at[pl.ds(r0, RPT)], acc.at[pl.ds(r0, RPT)])
    plsc.subcore_barrier()

    def body(j, carry):
        c0 = 4 * j
        for b in range(4):

            @pl.when(c0 + b < CPT)
            def _(b=b):
                pltpu.make_async_copy(
                    table.at[sslab.at[pl.ds((c0 + b) * K, K)]],
                    rows[b],
                    gsem[b],
                ).wait()
                pltpu.async_copy(
                    rows[b],
                    acc.at[dslab.at[pl.ds((c0 + b) * K, K)]],
                    ssem[b],
                    add=True,
                )

        for b in range(4):

            @pl.when(c0 + b + 4 < CPT)
            def _(b=b):
                pltpu.make_async_copy(
                    rows[b],
                    acc.at[dslab.at[pl.ds((c0 + b) * K, K)]],
                    ssem[b],
                ).wait()
                pltpu.async_copy(
                    table.at[sslab.at[pl.ds((c0 + b + 4) * K, K)]],
                    rows[b],
                    gsem[b],
                )

        return carry

    lax.fori_loop(0, (CPT + 3) // 4, body, 0)
    for b in range(4):
        pltpu.make_async_copy(
            rows[b], acc.at[dslab.at[pl.ds((CPT - 4 + b) * K, K)]], ssem[b]
        ).wait()
    plsc.subcore_barrier()
    pltpu.sync_copy(acc.at[pl.ds(r0, RPT)], out.at[ci, pl.ds(r0, RPT)])


BM = 1000


def _mm1_body(x_ref, w_ref, dg_ref, g_ref, dis_ref):
    deg = 1.0 + dg_ref[0, :, 0:1] + dg_ref[1, :, 0:1]
    dis = lax.rsqrt(deg)
    h = jnp.dot(x_ref[:], w_ref[:], preferred_element_type=jnp.float32)
    g_ref[:] = h * dis
    dis_ref[:] = dis


def _mid_body(g_ref, a_ref, dis_ref, b1_ref, o_ref):
    dis = dis_ref[:]
    z = (g_ref[:] + a_ref[0] + a_ref[1]) * dis + b1_ref[:]
    o_ref[:] = jnp.maximum(z, 0.0) * dis


def _mm2_body(g_ref, a_ref, dis_ref, w_ref, b2_ref, o_ref):
    u = (g_ref[:] + a_ref[0] + a_ref[1]) * dis_ref[:]
    o_ref[:] = (
        jnp.dot(u, w_ref[:], preferred_element_type=jnp.float32) + b2_ref[:]
    )


def kernel(x, edge_index, batch, W1, b1, W2, b2):
    del batch
    ones16 = jnp.ones((K, DEG_D), jnp.float32)
    zeros_deg = jnp.zeros((N_PAD, DEG_D), jnp.float32)
    zeros_hid = jnp.zeros((N_PAD, HID), jnp.float32)

    deg = _deg_kernel(edge_index, ones16, zeros_deg)

    g1, dis = pl.pallas_call(
        _mm1_body,
        grid=(N // BM,),
        in_specs=[
            pl.BlockSpec((BM, IN_D), lambda i: (i, 0)),
            pl.BlockSpec((IN_D, HID), lambda i: (0, 0)),
            pl.BlockSpec((NC, BM, DEG_D), lambda i: (0, i, 0)),
        ],
        out_specs=[
            pl.BlockSpec((BM, HID), lambda i: (i, 0)),
            pl.BlockSpec((BM, 1), lambda i: (i, 0)),
        ],
        out_shape=[
            jax.ShapeDtypeStruct((N, HID), jnp.float32),
            jax.ShapeDtypeStruct((N, 1), jnp.float32),
        ],
    )(x, W1, deg)

    acc1 = _prop_kernel(g1, edge_index, zeros_hid)

    g2 = pl.pallas_call(
        _mid_body,
        grid=(N // BM,),
        in_specs=[
            pl.BlockSpec((BM, HID), lambda i: (i, 0)),
            pl.BlockSpec((NC, BM, HID), lambda i: (0, i, 0)),
            pl.BlockSpec((BM, 1), lambda i: (i, 0)),
            pl.BlockSpec((1, HID), lambda i: (0, 0)),
        ],
        out_specs=pl.BlockSpec((BM, HID), lambda i: (i, 0)),
        out_shape=jax.ShapeDtypeStruct((N, HID), jnp.float32),
    )(g1, acc1, dis, b1.reshape(1, HID))

    acc2 = _prop_kernel(g2, edge_index, zeros_hid)

    out = pl.pallas_call(
        _mm2_body,
        grid=(N // BM,),
        in_specs=[
            pl.BlockSpec((BM, HID), lambda i: (i, 0)),
            pl.BlockSpec((NC, BM, HID), lambda i: (0, i, 0)),
            pl.BlockSpec((BM, 1), lambda i: (i, 0)),
            pl.BlockSpec((HID, OUT_D), lambda i: (0, 0)),
            pl.BlockSpec((1, OUT_D), lambda i: (0, 0)),
        ],
        out_specs=pl.BlockSpec((BM, OUT_D), lambda i: (i, 0)),
        out_shape=jax.ShapeDtypeStruct((N, OUT_D), jnp.float32),
    )(g2, acc2, dis, W2, b2.reshape(1, OUT_D))

    return out


# speedup vs baseline: 44.8035x; 1.0365x over previous
import functools
import jax
import jax.numpy as jnp
from jax import lax
from jax.experimental import pallas as pl
from jax.experimental.pallas import tpu as pltpu
from jax.experimental.pallas import tpu_sc as plsc

N = 10000
E = 320000
IN_D = 128
HID = 64
OUT_D = 116
NC, NS = 2, 16
NW = NC * NS
K = 200
CPT = 50
EPT = CPT * K
N_PAD = 10240
RPT = N_PAD // NS
DEG_D = 8

_mesh = plsc.VectorSubcoreMesh(
    core_axis_name="c", subcore_axis_name="s", num_cores=NC, num_subcores=NS
)


@functools.partial(
    pl.kernel,
    out_type=jax.ShapeDtypeStruct((NC, N_PAD, DEG_D), jnp.float32),
    mesh=_mesh,
    compiler_params=pltpu.CompilerParams(use_tc_tiling_on_sc=False),
    scratch_types=[
        pltpu.VMEM((EPT,), jnp.int32),
        pltpu.VMEM((K, DEG_D), jnp.float32),
        pltpu.VMEM_SHARED((N_PAD, DEG_D), jnp.float32),
    ],
)
def _deg_kernel(edges, ones_hbm, zeros_hbm, out, dslab, ones_v, acc):
    ci = lax.axis_index("c")
    si = lax.axis_index("s")
    wid = ci * NS + si
    r0 = si * RPT
    pltpu.sync_copy(ones_hbm, ones_v)
    pltpu.sync_copy(edges.at[1, pl.ds(wid * EPT, EPT)], dslab)
    pltpu.sync_copy(zeros_hbm.at[pl.ds(r0, RPT)], acc.at[pl.ds(r0, RPT)])
    plsc.subcore_barrier()

    def body(c, carry):
        pltpu.sync_copy(ones_v, acc.at[dslab.at[pl.ds(c * K, K)]], add=True)
        return carry

    lax.fori_loop(0, CPT, body, 0)
    plsc.subcore_barrier()
    pltpu.sync_copy(acc.at[pl.ds(r0, RPT)], out.at[ci, pl.ds(r0, RPT)])


@functools.partial(
    pl.kernel,
    out_type=jax.ShapeDtypeStruct((NC, N_PAD, HID), jnp.float32),
    mesh=_mesh,
    compiler_params=pltpu.CompilerParams(use_tc_tiling_on_sc=False),
    scratch_types=[
        pltpu.VMEM((EPT,), jnp.int32),
        pltpu.VMEM((EPT,), jnp.int32),
        [pltpu.VMEM((K, HID), jnp.float32)] * 4,
        pltpu.VMEM_SHARED((N_PAD, HID), jnp.float32),
        [pltpu.SemaphoreType.DMA] * 4,
        [pltpu.SemaphoreType.DMA] * 4,
    ],
)
def _prop_kernel(
    table, edges, zeros_hbm, out, sslab, dslab, rows, acc, gsem, ssem
):
    ci = lax.axis_index("c")
    si = lax.axis_index("s")
    wid = ci * NS + si
    r0 = si * RPT
    pltpu.sync_copy(edges.at[0, pl.ds(wid * EPT, EPT)], sslab)
    pltpu.sync_copy(edges.at[1, pl.ds(wid * EPT, EPT)], dslab)
    for b in range(4):
        pltpu.async_copy(
            table.at[sslab.at[pl.ds(b * K, K)]], rows[b], gsem[b]
        )
    pltpu.sync_copy(zeros_hbm.at[pl.ds(r0, RPT)], acc.at[pl.ds(r0, RPT)])
    plsc.subcore_barrier()

    def body(j, carry):
        c0 = 4 * j
        for b in range(4):

            @pl.when(c0 + b < CPT)
            def _(b=b):
                pltpu.make_async_copy(
                    table.at[sslab.at[pl.ds((c0 + b) * K, K)]],
                    rows[b],
                    gsem[b],
                ).wait()
                pltpu.async_copy(
                    rows[b],
                    acc.at[dslab.at[pl.ds((c0 + b) * K, K)]],
                    ssem[b],
                    add=True,
                )

        for b in range(4):

            @pl.when(c0 + b + 4 < CPT)
            def _(b=b):
                pltpu.make_async_copy(
                    rows[b],
                    acc.at[dslab.at[pl.ds((c0 + b) * K, K)]],
                    ssem[b],
                ).wait()
                pltpu.async_copy(
                    table.at[sslab.at[pl.ds((c0 + b + 4) * K, K)]],
                    rows[b],
                    gsem[b],
                )

        return carry

    lax.fori_loop(0, (CPT + 3) // 4, body, 0)
    for b in range(4):
        pltpu.make_async_copy(
            rows[b], acc.at[dslab.at[pl.ds((CPT - 4 + b) * K, K)]], ssem[b]
        ).wait()
    plsc.subcore_barrier()
    pltpu.sync_copy(acc.at[pl.ds(r0, RPT)], out.at[ci, pl.ds(r0, RPT)])


BM = 2000


def _mm1_body(x_ref, w_ref, dg_ref, g_ref, dis_ref):
    deg = 1.0 + dg_ref[0, :, 0:1] + dg_ref[1, :, 0:1]
    dis = lax.rsqrt(deg)
    h = jnp.dot(x_ref[:], w_ref[:], preferred_element_type=jnp.float32)
    g_ref[:] = h * dis
    dis_ref[:] = dis


def _mid_body(g_ref, a_ref, dis_ref, b1_ref, o_ref):
    dis = dis_ref[:]
    z = (g_ref[:] + a_ref[0] + a_ref[1]) * dis + b1_ref[:]
    o_ref[:] = jnp.maximum(z, 0.0) * dis


def _mm2_body(g_ref, a_ref, dis_ref, w_ref, b2_ref, o_ref):
    u = (g_ref[:] + a_ref[0] + a_ref[1]) * dis_ref[:]
    o_ref[:] = (
        jnp.dot(u, w_ref[:], preferred_element_type=jnp.float32) + b2_ref[:]
    )


def kernel(x, edge_index, batch, W1, b1, W2, b2):
    del batch
    ones16 = jnp.ones((K, DEG_D), jnp.float32)
    zeros_deg = jnp.zeros((N_PAD, DEG_D), jnp.float32)
    zeros_hid = jnp.zeros((N_PAD, HID), jnp.float32)

    deg = _deg_kernel(edge_index, ones16, zeros_deg)

    g1, dis = pl.pallas_call(
        _mm1_body,
        grid=(N // BM,),
        in_specs=[
            pl.BlockSpec((BM, IN_D), lambda i: (i, 0)),
            pl.BlockSpec((IN_D, HID), lambda i: (0, 0)),
            pl.BlockSpec((NC, BM, DEG_D), lambda i: (0, i, 0)),
        ],
        out_specs=[
            pl.BlockSpec((BM, HID), lambda i: (i, 0)),
            pl.BlockSpec((BM, 1), lambda i: (i, 0)),
        ],
        out_shape=[
            jax.ShapeDtypeStruct((N, HID), jnp.float32),
            jax.ShapeDtypeStruct((N, 1), jnp.float32),
        ],
    )(x, W1, deg)

    acc1 = _prop_kernel(g1, edge_index, zeros_hid)

    g2 = pl.pallas_call(
        _mid_body,
        grid=(N // BM,),
        in_specs=[
            pl.BlockSpec((BM, HID), lambda i: (i, 0)),
            pl.BlockSpec((NC, BM, HID), lambda i: (0, i, 0)),
            pl.BlockSpec((BM, 1), lambda i: (i, 0)),
            pl.BlockSpec((1, HID), lambda i: (0, 0)),
        ],
        out_specs=pl.BlockSpec((BM, HID), lambda i: (i, 0)),
        out_shape=jax.ShapeDtypeStruct((N, HID), jnp.float32),
    )(g1, acc1, dis, b1.reshape(1, HID))

    acc2 = _prop_kernel(g2, edge_index, zeros_hid)

    out = pl.pallas_call(
        _mm2_body,
        grid=(N // BM,),
        in_specs=[
            pl.BlockSpec((BM, HID), lambda i: (i, 0)),
            pl.BlockSpec((NC, BM, HID), lambda i: (0, i, 0)),
            pl.BlockSpec((BM, 1), lambda i: (i, 0)),
            pl.BlockSpec((HID, OUT_D), lambda i: (0, 0)),
            pl.BlockSpec((1, OUT_D), lambda i: (0, 0)),
        ],
        out_specs=pl.BlockSpec((BM, OUT_D), lambda i: (i, 0)),
        out_shape=jax.ShapeDtypeStruct((N, OUT_D), jnp.float32),
    )(g2, acc2, dis, W2, b2.reshape(1, OUT_D))

    return out
